# baseline (speedup 1.0000x reference)
import functools

import jax
import jax.numpy as jnp
import numpy as np
from jax import lax
from jax.experimental import pallas as pl
from jax.experimental.pallas import tpu as pltpu
from jax.experimental.pallas import tpu_sc as plsc

B = 2
H = 64
W = 64
N_LEVELS = 4
N_HEADS = 8
N_POINTS = 4
D_MODEL = 256
D_FFN = 1024
LQ = H * W
LIN = N_LEVELS * LQ
ROWS = B * LQ
DH = D_MODEL // N_HEADS
NLANE = N_HEADS * N_LEVELS * N_POINTS

_PERM = np.array([((m * N_LEVELS + l) * N_POINTS + p) * 2 + half
                  for half in range(2)
                  for m in range(N_HEADS)
                  for l in range(N_LEVELS)
                  for p in range(N_POINTS)], dtype=np.int32)

_lanes = np.arange(NLANE)
_lvl = (_lanes % (N_LEVELS * N_POINTS)) // N_POINTS
_EX = np.zeros((2 * N_LEVELS, NLANE), np.float32)
_EX[2 * _lvl, _lanes] = float(W)
_EY = np.zeros((2 * N_LEVELS, NLANE), np.float32)
_EY[2 * _lvl + 1, _lanes] = float(H)
_G = np.kron(np.eye(N_HEADS, dtype=np.float32),
             np.ones((N_LEVELS * N_POINTS, N_LEVELS * N_POINTS), np.float32))

_VAL_BLK = 1024


def _value_body(x, p, WvT, bv, out_ref):
    out_ref[...] = (jnp.dot(x[...] + p[0, 0], WvT[...],
                            preferred_element_type=jnp.float32,
                            precision=None) + bv[...])


def _value(src2d, pos4d, Wv, bv):
    n = B * LIN
    nl = LQ // _VAL_BLK
    row = pl.BlockSpec((_VAL_BLK, D_MODEL),
                       lambda b, l, i: (b * N_LEVELS * nl + l * nl + i, 0))
    pspec = pl.BlockSpec((1, 1, _VAL_BLK, D_MODEL), lambda b, l, i: (l, b, i, 0))
    full = lambda *s: pl.BlockSpec(s, lambda b, l, i: (0,) * len(s))
    return pl.pallas_call(
        _value_body,
        grid=(B, N_LEVELS, nl),
        in_specs=[row, pspec, full(D_MODEL, D_MODEL), full(D_MODEL)],
        out_specs=row,
        out_shape=jax.ShapeDtypeStruct((n, D_MODEL), jnp.float32),
    )(src2d, pos4d, Wv.T, bv)


_PREP_BLK = 512


def _prep_body(cur, posl, rp8, WofpT, bofp, WattT, batt, ex, ey, gmat,
               idx_ref, wt_ref):
    q = cur[...] + posl[...]
    off = jnp.dot(q, WofpT[...], preferred_element_type=jnp.float32, precision=lax.Precision.HIGHEST) + bofp[...]
    offx = off[:, :NLANE]
    offy = off[:, NLANE:]
    a = jnp.dot(q, WattT[...], preferred_element_type=jnp.float32, precision=lax.Precision.HIGHEST) + batt[...]
    a = a - jnp.max(a, axis=-1, keepdims=True)
    e = jnp.exp(a)
    s = jnp.dot(e, gmat[...], preferred_element_type=jnp.float32, precision=lax.Precision.HIGHEST)
    aw = e / s
    basex = jnp.dot(rp8[...], ex[...], preferred_element_type=jnp.float32, precision=lax.Precision.HIGHEST)
    basey = jnp.dot(rp8[...], ey[...], preferred_element_type=jnp.float32, precision=lax.Precision.HIGHEST)
    x = basex + offx - 0.5
    y = basey + offy - 0.5
    x0f = jnp.floor(x)
    y0f = jnp.floor(y)
    lx = x - x0f
    ly = y - y0f
    lane = lax.broadcasted_iota(jnp.int32, (_PREP_BLK, NLANE), 1)
    m_lane = lane >> 4
    l_lane = (lane >> 2) & 3
    b = pl.program_id(0) * _PREP_BLK // LQ
    base = (b * LIN + l_lane * LQ) * N_HEADS + m_lane
    x0 = x0f.astype(jnp.int32)
    y0 = y0f.astype(jnp.int32)
    x1 = x0 + 1
    y1 = y0 + 1
    vx0 = ((x0 >= 0) & (x0 <= W - 1)).astype(jnp.float32)
    vx1 = ((x1 >= 0) & (x1 <= W - 1)).astype(jnp.float32)
    vy0 = ((y0 >= 0) & (y0 <= H - 1)).astype(jnp.float32)
    vy1 = ((y1 >= 0) & (y1 <= H - 1)).astype(jnp.float32)
    xc0 = jnp.clip(x0, 0, W - 1)
    xc1 = jnp.clip(x1, 0, W - 1)
    yc0 = jnp.clip(y0, 0, H - 1)
    yc1 = jnp.clip(y1, 0, H - 1)
    taps = [
        (xc0, yc0, vx0 * vy0, (1.0 - lx) * (1.0 - ly)),
        (xc1, yc0, vx1 * vy0, lx * (1.0 - ly)),
        (xc0, yc1, vx0 * vy1, (1.0 - lx) * ly),
        (xc1, yc1, vx1 * vy1, lx * ly),
    ]
    for t, (xc, yc, v, bw) in enumerate(taps):
        idx_ref[t] = base + (yc * W + xc) * N_HEADS
        wt_ref[t] = aw * bw * v


def _prep(cur2d, posl2d, rp8, Wofp, bofp, Watt, batt):
    grid = ROWS // _PREP_BLK
    row = pl.BlockSpec((_PREP_BLK, D_MODEL), lambda i: (i, 0))
    row8 = pl.BlockSpec((_PREP_BLK, 2 * N_LEVELS), lambda i: (i, 0))
    out4 = pl.BlockSpec((4, _PREP_BLK, NLANE), lambda i: (0, i, 0))
    full = lambda *s: pl.BlockSpec(s, lambda i: (0,) * len(s))
    return pl.pallas_call(
        _prep_body,
        grid=(grid,),
        in_specs=[row, row, row8,
                  full(D_MODEL, D_MODEL), full(D_MODEL),
                  full(D_MODEL, NLANE), full(NLANE),
                  full(2 * N_LEVELS, NLANE), full(2 * N_LEVELS, NLANE),
                  full(NLANE, NLANE)],
        out_specs=(out4, out4),
        out_shape=(jax.ShapeDtypeStruct((4, ROWS, NLANE), jnp.int32),
                   jax.ShapeDtypeStruct((4, ROWS, NLANE), jnp.float32)),
    )(cur2d, posl2d, rp8, Wofp.T, bofp, Watt.T, batt,
      jnp.asarray(_EX), jnp.asarray(_EY), jnp.asarray(_G))


_NC = 2
_NS = 16
_NW = _NC * _NS
_RPW = ROWS // _NW
_SC_CH = 16
_NCH = _RPW // _SC_CH
_NCHP = _NCH // 2
_RING = 4

_GDN = lax.GatherDimensionNumbers(offset_dims=(), collapsed_slice_dims=(0,),
                                  start_index_map=(0,))


def _lane_bcast(v, j):
    idx = jnp.full((16, 1), j, jnp.int32)
    return lax.gather(v, idx, _GDN, (1,),
                      mode=lax.GatherScatterMode.PROMISE_IN_BOUNDS)


def _sc_body(base, nhalf, table, idx, wt, out, idx_v, wt_v, rows_v, out_v,
             sem_i, sem_g):
    rpw = nhalf // _NW
    nchp = rpw // _SC_CH // 2
    wid = lax.axis_index("s") * _NC + lax.axis_index("c")
    out0_w = wid * rpw
    row0_w = base + out0_w

    def chunk_dma(ch, cb):
        row0 = row0_w + ch * _SC_CH
        for t in range(4):
            pltpu.async_copy(idx.at[t, pl.ds(row0, _SC_CH)], idx_v.at[cb, t],
                             sem_i)
            pltpu.async_copy(wt.at[t, pl.ds(row0, _SC_CH)], wt_v.at[cb, t],
                             sem_i)

    def chunk_wait(ch, cb):
        row0 = row0_w + ch * _SC_CH
        for t in range(4):
            pltpu.make_async_copy(idx.at[t, pl.ds(row0, _SC_CH)],
                                  idx_v.at[cb, t], sem_i).wait()
            pltpu.make_async_copy(wt.at[t, pl.ds(row0, _SC_CH)],
                                  wt_v.at[cb, t], sem_i).wait()

    def issue(r, cb, buf):
        for t in range(4):
            pltpu.async_copy(table.at[idx_v.at[cb, t, r]], rows_v.at[buf, t],
                             sem_g.at[buf])

    def wait_gathers(r, cb, buf):
        for t in range(4):
            pltpu.make_async_copy(table.at[idx_v.at[cb, t, r]],
                                  rows_v.at[buf, t], sem_g.at[buf]).wait()

    def reduce_row(r, cb, buf):
        def m_body(m, carry):
            acc = [jnp.zeros((16,), jnp.float32) for _ in range(2)]
            for t in range(4):
                wv = wt_v[cb, t, r, pl.ds(m * 16, 16)]
                for j in range(16):
                    wb = _lane_bcast(wv, j)
                    for h in range(2):
                        vrow = rows_v[buf, t, m * 16 + j, pl.ds(h * 16, 16)]
                        acc[h] = acc[h] + wb * vrow
            out_v[r, pl.ds(m * DH, 16)] = acc[0]
            out_v[r, pl.ds(m * DH + 16, 16)] = acc[1]
            return carry
        lax.fori_loop(0, N_HEADS, m_body, 0)

    def chunk_pair_body(pr, carry):
        for cb in range(2):
            ch = 2 * pr + cb
            chunk_wait(ch, cb)
            for rr in range(_RING - 1):
                issue(rr, cb, rr)
            if cb == 0:
                chunk_dma(ch + 1, 1)
            else:
                @pl.when(pr < nchp - 1)
                def _():
                    chunk_dma(ch + 1, 0)

            def row_body(r, c2):
                buf = r & (_RING - 1)
                wait_gathers(r, cb, buf)

                @pl.when(r < _SC_CH - (_RING - 1))
                def _():
                    issue(r + _RING - 1, cb, (r + _RING - 1) & (_RING - 1))
                reduce_row(r, cb, buf)
                return c2

            lax.fori_loop(0, _SC_CH, row_body, 0)
            pltpu.sync_copy(out_v, out.at[pl.ds(out0_w + ch * _SC_CH, _SC_CH)])
        return carry

    chunk_dma(0, 0)
    lax.fori_loop(0, nchp, chunk_pair_body, 0)


@functools.cache
def _sc_attn_fn(base, nhalf):
    return pl.kernel(
        functools.partial(_sc_body, base, nhalf),
        out_type=jax.ShapeDtypeStruct((nhalf, D_MODEL), jnp.float32),
        mesh=plsc.VectorSubcoreMesh(core_axis_name="c", subcore_axis_name="s",
                                    num_cores=_NC, num_subcores=_NS),
        compiler_params=pltpu.CompilerParams(use_tc_tiling_on_sc=False),
        scratch_types=[
            pltpu.VMEM((2, 4, _SC_CH, NLANE), jnp.int32),
            pltpu.VMEM((2, 4, _SC_CH, NLANE), jnp.float32),
            pltpu.VMEM((_RING, 4, NLANE, DH), jnp.float32),
            pltpu.VMEM((_SC_CH, D_MODEL), jnp.float32),
            pltpu.SemaphoreType.DMA,
            pltpu.SemaphoreType.DMA((_RING,)),
        ],
    )


def _sc_attn(table, idx, wt, base, nhalf):
    return _sc_attn_fn(base, nhalf)(table, idx, wt)


_TAIL_BLK = 512


def _tail_body(attn, cur, WoutT, bout, W1T, b1, W2T, b2, g1, be1, g2, be2,
               out_ref):
    src2 = jnp.dot(attn[...], WoutT[...], preferred_element_type=jnp.float32, precision=None)
    x = cur[...] + src2 + bout[...]
    mu = jnp.mean(x, axis=-1, keepdims=True)
    var = jnp.mean((x - mu) ** 2, axis=-1, keepdims=True)
    x = (x - mu) * lax.rsqrt(var + 1e-5) * g1[...] + be1[...]
    h = jnp.dot(x, W1T[...], preferred_element_type=jnp.float32, precision=None) + b1[...]
    h = jnp.maximum(h, 0.0)
    y = jnp.dot(h, W2T[...], preferred_element_type=jnp.float32, precision=None) + b2[...]
    y = x + y
    mu = jnp.mean(y, axis=-1, keepdims=True)
    var = jnp.mean((y - mu) ** 2, axis=-1, keepdims=True)
    out_ref[...] = (y - mu) * lax.rsqrt(var + 1e-5) * g2[...] + be2[...]


def _tail(attn2d, cur2d, Wout, bout, W1, b1, W2, b2, g1, be1, g2, be2):
    nrows = attn2d.shape[0]
    grid = nrows // _TAIL_BLK
    row = pl.BlockSpec((_TAIL_BLK, D_MODEL), lambda i: (i, 0))
    full = lambda *s: pl.BlockSpec(s, lambda i: (0,) * len(s))
    return pl.pallas_call(
        _tail_body,
        grid=(grid,),
        in_specs=[row, row,
                  full(D_MODEL, D_MODEL), full(D_MODEL),
                  full(D_MODEL, D_FFN), full(D_FFN),
                  full(D_FFN, D_MODEL), full(D_MODEL),
                  full(D_MODEL), full(D_MODEL), full(D_MODEL), full(D_MODEL)],
        out_specs=row,
        out_shape=jax.ShapeDtypeStruct((nrows, D_MODEL), jnp.float32),
    )(attn2d, cur2d, Wout.T, bout, W1.T, b1, W2.T, b2, g1, be1, g2, be2)


def kernel(cur_src, src_all, pos, reference_points, spatial_shape,
           Wv, bv, Woff, boff, Watt, batt, Wout, bout,
           W1, b1, W2, b2, g1, be1, g2, be2):
    del spatial_shape
    value = _value(src_all.reshape(B * LIN, D_MODEL), pos, Wv, bv)
    table = value.reshape(B * LIN * N_HEADS, DH)

    perm = jnp.asarray(_PERM)
    idx, wt = _prep(cur_src.reshape(ROWS, D_MODEL),
                    pos[-1].reshape(ROWS, D_MODEL),
                    reference_points.reshape(ROWS, 2 * N_LEVELS),
                    Woff[perm], boff[perm], Watt, batt)

    nh = ROWS // 2
    cur2d = cur_src.reshape(ROWS, D_MODEL)
    outs = []
    for hidx in range(2):
        attn_h = _sc_attn(table, idx, wt, hidx * nh, nh)
        outs.append(_tail(attn_h, lax.slice_in_dim(cur2d, hidx * nh, (hidx + 1) * nh),
                          Wout, bout, W1, b1, W2, b2, g1, be1, g2, be2))
    return jnp.concatenate(outs, axis=0).reshape(B, LQ, D_MODEL)


# speedup vs baseline: 1.0678x; 1.0678x over previous
import functools

import jax
import jax.numpy as jnp
import numpy as np
from jax import lax
from jax.experimental import pallas as pl
from jax.experimental.pallas import tpu as pltpu
from jax.experimental.pallas import tpu_sc as plsc

B = 2
H = 64
W = 64
N_LEVELS = 4
N_HEADS = 8
N_POINTS = 4
D_MODEL = 256
D_FFN = 1024
LQ = H * W
LIN = N_LEVELS * LQ
ROWS = B * LQ
DH = D_MODEL // N_HEADS
NLANE = N_HEADS * N_LEVELS * N_POINTS

_PERM = np.array([((m * N_LEVELS + l) * N_POINTS + p) * 2 + half
                  for half in range(2)
                  for m in range(N_HEADS)
                  for l in range(N_LEVELS)
                  for p in range(N_POINTS)], dtype=np.int32)

_lanes = np.arange(NLANE)
_lvl = (_lanes % (N_LEVELS * N_POINTS)) // N_POINTS
_EX = np.zeros((2 * N_LEVELS, NLANE), np.float32)
_EX[2 * _lvl, _lanes] = float(W)
_EY = np.zeros((2 * N_LEVELS, NLANE), np.float32)
_EY[2 * _lvl + 1, _lanes] = float(H)
_G = np.kron(np.eye(N_HEADS, dtype=np.float32),
             np.ones((N_LEVELS * N_POINTS, N_LEVELS * N_POINTS), np.float32))

_VAL_BLK = 1024


def _value_body(x, p, WvT, bv, out_ref):
    out_ref[...] = (jnp.dot(x[...] + p[0, 0], WvT[...],
                            preferred_element_type=jnp.float32,
                            precision=None) + bv[...])


def _value(src2d, pos4d, Wv, bv):
    n = B * LIN
    nl = LQ // _VAL_BLK
    row = pl.BlockSpec((_VAL_BLK, D_MODEL),
                       lambda b, l, i: (b * N_LEVELS * nl + l * nl + i, 0))
    pspec = pl.BlockSpec((1, 1, _VAL_BLK, D_MODEL), lambda b, l, i: (l, b, i, 0))
    full = lambda *s: pl.BlockSpec(s, lambda b, l, i: (0,) * len(s))
    return pl.pallas_call(
        _value_body,
        grid=(B, N_LEVELS, nl),
        in_specs=[row, pspec, full(D_MODEL, D_MODEL), full(D_MODEL)],
        out_specs=row,
        out_shape=jax.ShapeDtypeStruct((n, D_MODEL), jnp.float32),
    )(src2d, pos4d, Wv.T, bv)


_PREP_BLK = 512


def _prep_body(cur, posl, rp8, WofpT, bofp, WattT, batt, ex, ey, gmat,
               idx_ref, wt_ref):
    q = cur[...] + posl[...]
    off = jnp.dot(q, WofpT[...], preferred_element_type=jnp.float32, precision=lax.Precision.HIGHEST) + bofp[...]
    offx = off[:, :NLANE]
    offy = off[:, NLANE:]
    a = jnp.dot(q, WattT[...], preferred_element_type=jnp.float32) + batt[...]
    a = a - jnp.max(a, axis=-1, keepdims=True)
    e = jnp.exp(a)
    s = jnp.dot(e, gmat[...], preferred_element_type=jnp.float32)
    aw = e / s
    basex = jnp.dot(rp8[...], ex[...], preferred_element_type=jnp.float32, precision=lax.Precision.HIGHEST)
    basey = jnp.dot(rp8[...], ey[...], preferred_element_type=jnp.float32, precision=lax.Precision.HIGHEST)
    x = basex + offx - 0.5
    y = basey + offy - 0.5
    x0f = jnp.floor(x)
    y0f = jnp.floor(y)
    lx = x - x0f
    ly = y - y0f
    lane = lax.broadcasted_iota(jnp.int32, (_PREP_BLK, NLANE), 1)
    m_lane = lane >> 4
    l_lane = (lane >> 2) & 3
    b = pl.program_id(0) * _PREP_BLK // LQ
    base = (b * LIN + l_lane * LQ) * N_HEADS + m_lane
    x0 = x0f.astype(jnp.int32)
    y0 = y0f.astype(jnp.int32)
    x1 = x0 + 1
    y1 = y0 + 1
    vx0 = ((x0 >= 0) & (x0 <= W - 1)).astype(jnp.float32)
    vx1 = ((x1 >= 0) & (x1 <= W - 1)).astype(jnp.float32)
    vy0 = ((y0 >= 0) & (y0 <= H - 1)).astype(jnp.float32)
    vy1 = ((y1 >= 0) & (y1 <= H - 1)).astype(jnp.float32)
    xc0 = jnp.clip(x0, 0, W - 1)
    xc1 = jnp.clip(x1, 0, W - 1)
    yc0 = jnp.clip(y0, 0, H - 1)
    yc1 = jnp.clip(y1, 0, H - 1)
    taps = [
        (xc0, yc0, vx0 * vy0, (1.0 - lx) * (1.0 - ly)),
        (xc1, yc0, vx1 * vy0, lx * (1.0 - ly)),
        (xc0, yc1, vx0 * vy1, (1.0 - lx) * ly),
        (xc1, yc1, vx1 * vy1, lx * ly),
    ]
    for t, (xc, yc, v, bw) in enumerate(taps):
        idx_ref[t] = base + (yc * W + xc) * N_HEADS
        wt_ref[t] = aw * bw * v


def _prep(cur2d, posl2d, rp8, Wofp, bofp, Watt, batt):
    grid = ROWS // _PREP_BLK
    row = pl.BlockSpec((_PREP_BLK, D_MODEL), lambda i: (i, 0))
    row8 = pl.BlockSpec((_PREP_BLK, 2 * N_LEVELS), lambda i: (i, 0))
    out4 = pl.BlockSpec((4, _PREP_BLK, NLANE), lambda i: (0, i, 0))
    full = lambda *s: pl.BlockSpec(s, lambda i: (0,) * len(s))
    return pl.pallas_call(
        _prep_body,
        grid=(grid,),
        in_specs=[row, row, row8,
                  full(D_MODEL, D_MODEL), full(D_MODEL),
                  full(D_MODEL, NLANE), full(NLANE),
                  full(2 * N_LEVELS, NLANE), full(2 * N_LEVELS, NLANE),
                  full(NLANE, NLANE)],
        out_specs=(out4, out4),
        out_shape=(jax.ShapeDtypeStruct((4, ROWS, NLANE), jnp.int32),
                   jax.ShapeDtypeStruct((4, ROWS, NLANE), jnp.float32)),
    )(cur2d, posl2d, rp8, Wofp.T, bofp, Watt.T, batt,
      jnp.asarray(_EX), jnp.asarray(_EY), jnp.asarray(_G))


_NC = 2
_NS = 16
_NW = _NC * _NS
_RPW = ROWS // _NW
_SC_CH = 16
_NCH = _RPW // _SC_CH
_NCHP = _NCH // 2
_RING = 4

_GDN = lax.GatherDimensionNumbers(offset_dims=(), collapsed_slice_dims=(0,),
                                  start_index_map=(0,))


def _lane_bcast(v, j):
    idx = jnp.full((16, 1), j, jnp.int32)
    return lax.gather(v, idx, _GDN, (1,),
                      mode=lax.GatherScatterMode.PROMISE_IN_BOUNDS)


def _sc_body(base, nhalf, table, idx, wt, out, idx_v, wt_v, rows_v, out_v,
             sem_i, sem_g):
    rpw = nhalf // _NW
    nchp = rpw // _SC_CH // 2
    wid = lax.axis_index("s") * _NC + lax.axis_index("c")
    out0_w = wid * rpw
    row0_w = base + out0_w

    def chunk_dma(ch, cb):
        row0 = row0_w + ch * _SC_CH
        for t in range(4):
            pltpu.async_copy(idx.at[t, pl.ds(row0, _SC_CH)], idx_v.at[cb, t],
                             sem_i)
            pltpu.async_copy(wt.at[t, pl.ds(row0, _SC_CH)], wt_v.at[cb, t],
                             sem_i)

    def chunk_wait(ch, cb):
        row0 = row0_w + ch * _SC_CH
        for t in range(4):
            pltpu.make_async_copy(idx.at[t, pl.ds(row0, _SC_CH)],
                                  idx_v.at[cb, t], sem_i).wait()
            pltpu.make_async_copy(wt.at[t, pl.ds(row0, _SC_CH)],
                                  wt_v.at[cb, t], sem_i).wait()

    def issue(r, cb, buf):
        for t in range(4):
            pltpu.async_copy(table.at[idx_v.at[cb, t, r]], rows_v.at[buf, t],
                             sem_g.at[buf])

    def wait_gathers(r, cb, buf):
        for t in range(4):
            pltpu.make_async_copy(table.at[idx_v.at[cb, t, r]],
                                  rows_v.at[buf, t], sem_g.at[buf]).wait()

    def reduce_row(r, cb, buf):
        def m_body(m, carry):
            acc = [jnp.zeros((16,), jnp.float32) for _ in range(2)]
            for t in range(4):
                wv = wt_v[cb, t, r, pl.ds(m * 16, 16)]
                for j in range(16):
                    wb = _lane_bcast(wv, j)
                    for h in range(2):
                        vrow = rows_v[buf, t, m * 16 + j, pl.ds(h * 16, 16)]
                        acc[h] = acc[h] + wb * vrow
            out_v[r, pl.ds(m * DH, 16)] = acc[0]
            out_v[r, pl.ds(m * DH + 16, 16)] = acc[1]
            return carry
        lax.fori_loop(0, N_HEADS, m_body, 0)

    def chunk_pair_body(pr, carry):
        for cb in range(2):
            ch = 2 * pr + cb
            chunk_wait(ch, cb)
            for rr in range(_RING - 1):
                issue(rr, cb, rr)
            if cb == 0:
                chunk_dma(ch + 1, 1)
            else:
                @pl.when(pr < nchp - 1)
                def _():
                    chunk_dma(ch + 1, 0)

            def row_body(r, c2):
                buf = r & (_RING - 1)
                wait_gathers(r, cb, buf)

                @pl.when(r < _SC_CH - (_RING - 1))
                def _():
                    issue(r + _RING - 1, cb, (r + _RING - 1) & (_RING - 1))
                reduce_row(r, cb, buf)
                return c2

            lax.fori_loop(0, _SC_CH, row_body, 0)
            pltpu.sync_copy(out_v, out.at[pl.ds(out0_w + ch * _SC_CH, _SC_CH)])
        return carry

    chunk_dma(0, 0)
    lax.fori_loop(0, nchp, chunk_pair_body, 0)


@functools.cache
def _sc_attn_fn(base, nhalf):
    return pl.kernel(
        functools.partial(_sc_body, base, nhalf),
        out_type=jax.ShapeDtypeStruct((nhalf, D_MODEL), jnp.float32),
        mesh=plsc.VectorSubcoreMesh(core_axis_name="c", subcore_axis_name="s",
                                    num_cores=_NC, num_subcores=_NS),
        compiler_params=pltpu.CompilerParams(use_tc_tiling_on_sc=False),
        scratch_types=[
            pltpu.VMEM((2, 4, _SC_CH, NLANE), jnp.int32),
            pltpu.VMEM((2, 4, _SC_CH, NLANE), jnp.float32),
            pltpu.VMEM((_RING, 4, NLANE, DH), jnp.float32),
            pltpu.VMEM((_SC_CH, D_MODEL), jnp.float32),
            pltpu.SemaphoreType.DMA,
            pltpu.SemaphoreType.DMA((_RING,)),
        ],
    )


def _sc_attn(table, idx, wt, base, nhalf):
    return _sc_attn_fn(base, nhalf)(table, idx, wt)


_TAIL_BLK = 512


def _tail_body(attn, cur, WoutT, bout, W1T, b1, W2T, b2, g1, be1, g2, be2,
               out_ref):
    src2 = jnp.dot(attn[...], WoutT[...], preferred_element_type=jnp.float32, precision=None)
    x = cur[...] + src2 + bout[...]
    mu = jnp.mean(x, axis=-1, keepdims=True)
    var = jnp.mean((x - mu) ** 2, axis=-1, keepdims=True)
    x = (x - mu) * lax.rsqrt(var + 1e-5) * g1[...] + be1[...]
    h = jnp.dot(x, W1T[...], preferred_element_type=jnp.float32, precision=None) + b1[...]
    h = jnp.maximum(h, 0.0)
    y = jnp.dot(h, W2T[...], preferred_element_type=jnp.float32, precision=None) + b2[...]
    y = x + y
    mu = jnp.mean(y, axis=-1, keepdims=True)
    var = jnp.mean((y - mu) ** 2, axis=-1, keepdims=True)
    out_ref[...] = (y - mu) * lax.rsqrt(var + 1e-5) * g2[...] + be2[...]


def _tail(attn2d, cur2d, Wout, bout, W1, b1, W2, b2, g1, be1, g2, be2):
    nrows = attn2d.shape[0]
    grid = nrows // _TAIL_BLK
    row = pl.BlockSpec((_TAIL_BLK, D_MODEL), lambda i: (i, 0))
    full = lambda *s: pl.BlockSpec(s, lambda i: (0,) * len(s))
    return pl.pallas_call(
        _tail_body,
        grid=(grid,),
        in_specs=[row, row,
                  full(D_MODEL, D_MODEL), full(D_MODEL),
                  full(D_MODEL, D_FFN), full(D_FFN),
                  full(D_FFN, D_MODEL), full(D_MODEL),
                  full(D_MODEL), full(D_MODEL), full(D_MODEL), full(D_MODEL)],
        out_specs=row,
        out_shape=jax.ShapeDtypeStruct((nrows, D_MODEL), jnp.float32),
    )(attn2d, cur2d, Wout.T, bout, W1.T, b1, W2.T, b2, g1, be1, g2, be2)


def kernel(cur_src, src_all, pos, reference_points, spatial_shape,
           Wv, bv, Woff, boff, Watt, batt, Wout, bout,
           W1, b1, W2, b2, g1, be1, g2, be2):
    del spatial_shape
    value = _value(src_all.reshape(B * LIN, D_MODEL), pos, Wv, bv)
    table = value.reshape(B * LIN * N_HEADS, DH)

    perm = jnp.asarray(_PERM)
    idx, wt = _prep(cur_src.reshape(ROWS, D_MODEL),
                    pos[-1].reshape(ROWS, D_MODEL),
                    reference_points.reshape(ROWS, 2 * N_LEVELS),
                    Woff[perm], boff[perm], Watt, batt)

    attn2d = _sc_attn(table, idx, wt, 0, ROWS)
    out = _tail(attn2d, cur_src.reshape(ROWS, D_MODEL),
                Wout, bout, W1, b1, W2, b2, g1, be1, g2, be2)
    return out.reshape(B, LQ, D_MODEL)


# speedup vs baseline: 1.1156x; 1.0448x over previous
import functools

import jax
import jax.numpy as jnp
import numpy as np
from jax import lax
from jax.experimental import pallas as pl
from jax.experimental.pallas import tpu as pltpu
from jax.experimental.pallas import tpu_sc as plsc

B = 2
H = 64
W = 64
N_LEVELS = 4
N_HEADS = 8
N_POINTS = 4
D_MODEL = 256
D_FFN = 1024
LQ = H * W
LIN = N_LEVELS * LQ
ROWS = B * LQ
DH = D_MODEL // N_HEADS
NLANE = N_HEADS * N_LEVELS * N_POINTS

_PERM = np.array([((m * N_LEVELS + l) * N_POINTS + p) * 2 + half
                  for half in range(2)
                  for m in range(N_HEADS)
                  for l in range(N_LEVELS)
                  for p in range(N_POINTS)], dtype=np.int32)

_lanes = np.arange(NLANE)
_lvl = (_lanes % (N_LEVELS * N_POINTS)) // N_POINTS
_EX = np.zeros((2 * N_LEVELS, NLANE), np.float32)
_EX[2 * _lvl, _lanes] = float(W)
_EY = np.zeros((2 * N_LEVELS, NLANE), np.float32)
_EY[2 * _lvl + 1, _lanes] = float(H)
_G = np.kron(np.eye(N_HEADS, dtype=np.float32),
             np.ones((N_LEVELS * N_POINTS, N_LEVELS * N_POINTS), np.float32))

_VAL_BLK = 1024


def _value_body(x, p, WvT, bv, out_ref):
    out_ref[...] = (jnp.dot(x[...] + p[0, 0], WvT[...],
                            preferred_element_type=jnp.float32,
                            precision=None) + bv[...]).astype(jnp.bfloat16)


def _value(src2d, pos4d, Wv, bv):
    n = B * LIN
    nl = LQ // _VAL_BLK
    row = pl.BlockSpec((_VAL_BLK, D_MODEL),
                       lambda b, l, i: (b * N_LEVELS * nl + l * nl + i, 0))
    pspec = pl.BlockSpec((1, 1, _VAL_BLK, D_MODEL), lambda b, l, i: (l, b, i, 0))
    full = lambda *s: pl.BlockSpec(s, lambda b, l, i: (0,) * len(s))
    return pl.pallas_call(
        _value_body,
        grid=(B, N_LEVELS, nl),
        in_specs=[row, pspec, full(D_MODEL, D_MODEL), full(D_MODEL)],
        out_specs=row,
        out_shape=jax.ShapeDtypeStruct((n, D_MODEL), jnp.bfloat16),
    )(src2d, pos4d, Wv.T, bv)


_PREP_BLK = 512


def _prep_body(cur, posl, rp8, WofpT, bofp, WattT, batt, ex, ey, gmat,
               idx_ref, wt_ref):
    q = cur[...] + posl[...]
    off = jnp.dot(q, WofpT[...], preferred_element_type=jnp.float32, precision=lax.Precision.HIGHEST) + bofp[...]
    offx = off[:, :NLANE]
    offy = off[:, NLANE:]
    a = jnp.dot(q, WattT[...], preferred_element_type=jnp.float32) + batt[...]
    a = a - jnp.max(a, axis=-1, keepdims=True)
    e = jnp.exp(a)
    s = jnp.dot(e, gmat[...], preferred_element_type=jnp.float32)
    aw = e / s
    basex = jnp.dot(rp8[...], ex[...], preferred_element_type=jnp.float32, precision=lax.Precision.HIGHEST)
    basey = jnp.dot(rp8[...], ey[...], preferred_element_type=jnp.float32, precision=lax.Precision.HIGHEST)
    x = basex + offx - 0.5
    y = basey + offy - 0.5
    x0f = jnp.floor(x)
    y0f = jnp.floor(y)
    lx = x - x0f
    ly = y - y0f
    lane = lax.broadcasted_iota(jnp.int32, (_PREP_BLK, NLANE), 1)
    m_lane = lane >> 4
    l_lane = (lane >> 2) & 3
    b = pl.program_id(0) * _PREP_BLK // LQ
    base = (b * LIN + l_lane * LQ) * N_HEADS + m_lane
    x0 = x0f.astype(jnp.int32)
    y0 = y0f.astype(jnp.int32)
    x1 = x0 + 1
    y1 = y0 + 1
    vx0 = ((x0 >= 0) & (x0 <= W - 1)).astype(jnp.float32)
    vx1 = ((x1 >= 0) & (x1 <= W - 1)).astype(jnp.float32)
    vy0 = ((y0 >= 0) & (y0 <= H - 1)).astype(jnp.float32)
    vy1 = ((y1 >= 0) & (y1 <= H - 1)).astype(jnp.float32)
    xc0 = jnp.clip(x0, 0, W - 1)
    xc1 = jnp.clip(x1, 0, W - 1)
    yc0 = jnp.clip(y0, 0, H - 1)
    yc1 = jnp.clip(y1, 0, H - 1)
    taps = [
        (xc0, yc0, vx0 * vy0, (1.0 - lx) * (1.0 - ly)),
        (xc1, yc0, vx1 * vy0, lx * (1.0 - ly)),
        (xc0, yc1, vx0 * vy1, (1.0 - lx) * ly),
        (xc1, yc1, vx1 * vy1, lx * ly),
    ]
    for t, (xc, yc, v, bw) in enumerate(taps):
        idx_ref[t] = base + (yc * W + xc) * N_HEADS
        w16 = lax.bitcast_convert_type((aw * bw * v).astype(jnp.bfloat16),
                                       jnp.uint16).astype(jnp.uint32)
        wt_ref[t] = lax.bitcast_convert_type((w16 << 16) | w16, jnp.int32)


def _prep(cur2d, posl2d, rp8, Wofp, bofp, Watt, batt):
    grid = ROWS // _PREP_BLK
    row = pl.BlockSpec((_PREP_BLK, D_MODEL), lambda i: (i, 0))
    row8 = pl.BlockSpec((_PREP_BLK, 2 * N_LEVELS), lambda i: (i, 0))
    out4 = pl.BlockSpec((4, _PREP_BLK, NLANE), lambda i: (0, i, 0))
    full = lambda *s: pl.BlockSpec(s, lambda i: (0,) * len(s))
    return pl.pallas_call(
        _prep_body,
        grid=(grid,),
        in_specs=[row, row, row8,
                  full(D_MODEL, D_MODEL), full(D_MODEL),
                  full(D_MODEL, NLANE), full(NLANE),
                  full(2 * N_LEVELS, NLANE), full(2 * N_LEVELS, NLANE),
                  full(NLANE, NLANE)],
        out_specs=(out4, out4),
        out_shape=(jax.ShapeDtypeStruct((4, ROWS, NLANE), jnp.int32),
                   jax.ShapeDtypeStruct((4, ROWS, NLANE), jnp.int32)),
    )(cur2d, posl2d, rp8, Wofp.T, bofp, Watt.T, batt,
      jnp.asarray(_EX), jnp.asarray(_EY), jnp.asarray(_G))


_NC = 2
_NS = 16
_NW = _NC * _NS
_RPW = ROWS // _NW
_SC_CH = 16
_NCH = _RPW // _SC_CH
_NCHP = _NCH // 2
_RING = 8

_GDN = lax.GatherDimensionNumbers(offset_dims=(), collapsed_slice_dims=(0,),
                                  start_index_map=(0,))


def _lane_bcast(v, j):
    idx = jnp.full((16, 1), j, jnp.int32)
    return lax.gather(v, idx, _GDN, (1,),
                      mode=lax.GatherScatterMode.PROMISE_IN_BOUNDS)


def _sc_body(base, nhalf, table, idx, wt, out, idx_v, wt_v, rows_v, out_v,
             sem_i, sem_g):
    rpw = nhalf // _NW
    nchp = rpw // _SC_CH // 2
    wid = lax.axis_index("s") * _NC + lax.axis_index("c")
    out0_w = wid * rpw
    row0_w = base + out0_w

    def chunk_dma(ch, cb):
        row0 = row0_w + ch * _SC_CH
        for t in range(4):
            pltpu.async_copy(idx.at[t, pl.ds(row0, _SC_CH)], idx_v.at[cb, t],
                             sem_i)
            pltpu.async_copy(wt.at[t, pl.ds(row0, _SC_CH)], wt_v.at[cb, t],
                             sem_i)

    def chunk_wait(ch, cb):
        row0 = row0_w + ch * _SC_CH
        for t in range(4):
            pltpu.make_async_copy(idx.at[t, pl.ds(row0, _SC_CH)],
                                  idx_v.at[cb, t], sem_i).wait()
            pltpu.make_async_copy(wt.at[t, pl.ds(row0, _SC_CH)],
                                  wt_v.at[cb, t], sem_i).wait()

    def issue(r, cb, buf):
        for t in range(4):
            pltpu.async_copy(table.at[idx_v.at[cb, t, r]], rows_v.at[buf, t],
                             sem_g.at[buf])

    def wait_gathers(r, cb, buf):
        for t in range(4):
            pltpu.make_async_copy(table.at[idx_v.at[cb, t, r]],
                                  rows_v.at[buf, t], sem_g.at[buf]).wait()

    def reduce_row(r, cb, buf):
        lane2 = 2 * lax.iota(jnp.int32, 16)

        def m_body(m, carry):
            acc_e = jnp.zeros((16,), jnp.float32)
            acc_o = jnp.zeros((16,), jnp.float32)
            for t in range(4):
                wv = wt_v[cb, t, r, pl.ds(m * 16, 16)]
                for j in range(16):
                    wb = plsc.bitcast(_lane_bcast(wv, j), jnp.bfloat16)
                    prod = wb * rows_v[buf, t, m * 16 + j, :]
                    pe, po = plsc.unpack(prod,
                                         format=plsc.PackFormat.INTERLEAVED)
                    acc_e = acc_e + pe
                    acc_o = acc_o + po
            rr = jnp.full((16,), r, jnp.int32)
            plsc.store_scatter(out_v, [rr, m * DH + lane2], acc_e)
            plsc.store_scatter(out_v, [rr, m * DH + lane2 + 1], acc_o)
            return carry
        lax.fori_loop(0, N_HEADS, m_body, 0)

    def chunk_pair_body(pr, carry):
        for cb in range(2):
            ch = 2 * pr + cb
            chunk_wait(ch, cb)
            for rr in range(_RING - 1):
                issue(rr, cb, rr)
            if cb == 0:
                chunk_dma(ch + 1, 1)
            else:
                @pl.when(pr < nchp - 1)
                def _():
                    chunk_dma(ch + 1, 0)

            def row_body(r, c2):
                buf = r & (_RING - 1)
                wait_gathers(r, cb, buf)

                @pl.when(r < _SC_CH - (_RING - 1))
                def _():
                    issue(r + _RING - 1, cb, (r + _RING - 1) & (_RING - 1))
                reduce_row(r, cb, buf)
                return c2

            lax.fori_loop(0, _SC_CH, row_body, 0)
            pltpu.sync_copy(out_v, out.at[pl.ds(out0_w + ch * _SC_CH, _SC_CH)])
        return carry

    chunk_dma(0, 0)
    lax.fori_loop(0, nchp, chunk_pair_body, 0)


@functools.cache
def _sc_attn_fn(base, nhalf):
    return pl.kernel(
        functools.partial(_sc_body, base, nhalf),
        out_type=jax.ShapeDtypeStruct((nhalf, D_MODEL), jnp.float32),
        mesh=plsc.VectorSubcoreMesh(core_axis_name="c", subcore_axis_name="s",
                                    num_cores=_NC, num_subcores=_NS),
        compiler_params=pltpu.CompilerParams(use_tc_tiling_on_sc=False,
                                             needs_layout_passes=False),
        scratch_types=[
            pltpu.VMEM((2, 4, _SC_CH, NLANE), jnp.int32),
            pltpu.VMEM((2, 4, _SC_CH, NLANE), jnp.int32),
            pltpu.VMEM((_RING, 4, NLANE, DH), jnp.bfloat16),
            pltpu.VMEM((_SC_CH, D_MODEL), jnp.float32),
            pltpu.SemaphoreType.DMA,
            pltpu.SemaphoreType.DMA((_RING,)),
        ],
    )


def _sc_attn(table, idx, wt, base, nhalf):
    return _sc_attn_fn(base, nhalf)(table, idx, wt)


_TAIL_BLK = 512


def _tail_body(attn, cur, WoutT, bout, W1T, b1, W2T, b2, g1, be1, g2, be2,
               out_ref):
    src2 = jnp.dot(attn[...], WoutT[...], preferred_element_type=jnp.float32, precision=None)
    x = cur[...] + src2 + bout[...]
    mu = jnp.mean(x, axis=-1, keepdims=True)
    var = jnp.mean((x - mu) ** 2, axis=-1, keepdims=True)
    x = (x - mu) * lax.rsqrt(var + 1e-5) * g1[...] + be1[...]
    h = jnp.dot(x, W1T[...], preferred_element_type=jnp.float32, precision=None) + b1[...]
    h = jnp.maximum(h, 0.0)
    y = jnp.dot(h, W2T[...], preferred_element_type=jnp.float32, precision=None) + b2[...]
    y = x + y
    mu = jnp.mean(y, axis=-1, keepdims=True)
    var = jnp.mean((y - mu) ** 2, axis=-1, keepdims=True)
    out_ref[...] = (y - mu) * lax.rsqrt(var + 1e-5) * g2[...] + be2[...]


def _tail(attn2d, cur2d, Wout, bout, W1, b1, W2, b2, g1, be1, g2, be2):
    nrows = attn2d.shape[0]
    grid = nrows // _TAIL_BLK
    row = pl.BlockSpec((_TAIL_BLK, D_MODEL), lambda i: (i, 0))
    full = lambda *s: pl.BlockSpec(s, lambda i: (0,) * len(s))
    return pl.pallas_call(
        _tail_body,
        grid=(grid,),
        in_specs=[row, row,
                  full(D_MODEL, D_MODEL), full(D_MODEL),
                  full(D_MODEL, D_FFN), full(D_FFN),
                  full(D_FFN, D_MODEL), full(D_MODEL),
                  full(D_MODEL), full(D_MODEL), full(D_MODEL), full(D_MODEL)],
        out_specs=row,
        out_shape=jax.ShapeDtypeStruct((nrows, D_MODEL), jnp.float32),
    )(attn2d, cur2d, Wout.T, bout, W1.T, b1, W2.T, b2, g1, be1, g2, be2)


def kernel(cur_src, src_all, pos, reference_points, spatial_shape,
           Wv, bv, Woff, boff, Watt, batt, Wout, bout,
           W1, b1, W2, b2, g1, be1, g2, be2):
    del spatial_shape
    value = _value(src_all.reshape(B * LIN, D_MODEL), pos, Wv, bv)
    table = value.reshape(B * LIN * N_HEADS, DH)

    perm = jnp.asarray(_PERM)
    idx, wt = _prep(cur_src.reshape(ROWS, D_MODEL),
                    pos[-1].reshape(ROWS, D_MODEL),
                    reference_points.reshape(ROWS, 2 * N_LEVELS),
                    Woff[perm], boff[perm], Watt, batt)

    attn2d = _sc_attn(table, idx, wt, 0, ROWS)
    out = _tail(attn2d, cur_src.reshape(ROWS, D_MODEL),
                Wout, bout, W1, b1, W2, b2, g1, be1, g2, be2)
    return out.reshape(B, LQ, D_MODEL)


# speedup vs baseline: 1.1236x; 1.0072x over previous
import functools

import jax
import jax.numpy as jnp
import numpy as np
from jax import lax
from jax.experimental import pallas as pl
from jax.experimental.pallas import tpu as pltpu
from jax.experimental.pallas import tpu_sc as plsc

B = 2
H = 64
W = 64
N_LEVELS = 4
N_HEADS = 8
N_POINTS = 4
D_MODEL = 256
D_FFN = 1024
LQ = H * W
LIN = N_LEVELS * LQ
ROWS = B * LQ
DH = D_MODEL // N_HEADS
NLANE = N_HEADS * N_LEVELS * N_POINTS

_PERM = np.array([((m * N_LEVELS + l) * N_POINTS + p) * 2 + half
                  for half in range(2)
                  for m in range(N_HEADS)
                  for l in range(N_LEVELS)
                  for p in range(N_POINTS)], dtype=np.int32)

_lanes = np.arange(NLANE)
_lvl = (_lanes % (N_LEVELS * N_POINTS)) // N_POINTS
_EX = np.zeros((2 * N_LEVELS, NLANE), np.float32)
_EX[2 * _lvl, _lanes] = float(W)
_EY = np.zeros((2 * N_LEVELS, NLANE), np.float32)
_EY[2 * _lvl + 1, _lanes] = float(H)
_G = np.kron(np.eye(N_HEADS, dtype=np.float32),
             np.ones((N_LEVELS * N_POINTS, N_LEVELS * N_POINTS), np.float32))

_VAL_BLK = 1024


def _value_body(x, p, WvT, bv, out_ref):
    out_ref[...] = (jnp.dot(x[...] + p[0, 0], WvT[...],
                            preferred_element_type=jnp.float32,
                            precision=None) + bv[...]).astype(jnp.bfloat16)


def _value(src2d, pos4d, Wv, bv):
    n = B * LIN
    nl = LQ // _VAL_BLK
    row = pl.BlockSpec((_VAL_BLK, D_MODEL),
                       lambda b, l, i: (b * N_LEVELS * nl + l * nl + i, 0))
    pspec = pl.BlockSpec((1, 1, _VAL_BLK, D_MODEL), lambda b, l, i: (l, b, i, 0))
    full = lambda *s: pl.BlockSpec(s, lambda b, l, i: (0,) * len(s))
    return pl.pallas_call(
        _value_body,
        grid=(B, N_LEVELS, nl),
        in_specs=[row, pspec, full(D_MODEL, D_MODEL), full(D_MODEL)],
        out_specs=row,
        out_shape=jax.ShapeDtypeStruct((n, D_MODEL), jnp.bfloat16),
    )(src2d, pos4d, Wv.T, bv)


_PREP_BLK = 512


def _prep_body(cur, posl, rp8, WofpT, bofp, WattT, batt, ex, ey, gmat,
               idx_ref, wt_ref):
    q = cur[...] + posl[...]
    w = WofpT[...]
    whi = w.astype(jnp.bfloat16)
    wlo = (w - whi.astype(jnp.float32)).astype(jnp.bfloat16)
    qhi = q.astype(jnp.bfloat16)
    qlo = (q - qhi.astype(jnp.float32)).astype(jnp.bfloat16)
    off = (jnp.dot(qhi, whi, preferred_element_type=jnp.float32)
           + jnp.dot(qhi, wlo, preferred_element_type=jnp.float32)
           + jnp.dot(qlo, whi, preferred_element_type=jnp.float32)
           + bofp[...])
    offx = off[:, :NLANE]
    offy = off[:, NLANE:]
    a = jnp.dot(q, WattT[...], preferred_element_type=jnp.float32) + batt[...]
    a = a - jnp.max(a, axis=-1, keepdims=True)
    e = jnp.exp(a)
    s = jnp.dot(e, gmat[...], preferred_element_type=jnp.float32)
    aw = e / s
    basex = jnp.dot(rp8[...], ex[...], preferred_element_type=jnp.float32, precision=lax.Precision.HIGHEST)
    basey = jnp.dot(rp8[...], ey[...], preferred_element_type=jnp.float32, precision=lax.Precision.HIGHEST)
    x = basex + offx - 0.5
    y = basey + offy - 0.5
    x0f = jnp.floor(x)
    y0f = jnp.floor(y)
    lx = x - x0f
    ly = y - y0f
    lane = lax.broadcasted_iota(jnp.int32, (_PREP_BLK, NLANE), 1)
    m_lane = lane >> 4
    l_lane = (lane >> 2) & 3
    b = pl.program_id(0) * _PREP_BLK // LQ
    base = (b * LIN + l_lane * LQ) * N_HEADS + m_lane
    x0 = x0f.astype(jnp.int32)
    y0 = y0f.astype(jnp.int32)
    x1 = x0 + 1
    y1 = y0 + 1
    vx0 = ((x0 >= 0) & (x0 <= W - 1)).astype(jnp.float32)
    vx1 = ((x1 >= 0) & (x1 <= W - 1)).astype(jnp.float32)
    vy0 = ((y0 >= 0) & (y0 <= H - 1)).astype(jnp.float32)
    vy1 = ((y1 >= 0) & (y1 <= H - 1)).astype(jnp.float32)
    xc0 = jnp.clip(x0, 0, W - 1)
    xc1 = jnp.clip(x1, 0, W - 1)
    yc0 = jnp.clip(y0, 0, H - 1)
    yc1 = jnp.clip(y1, 0, H - 1)
    taps = [
        (xc0, yc0, vx0 * vy0, (1.0 - lx) * (1.0 - ly)),
        (xc1, yc0, vx1 * vy0, lx * (1.0 - ly)),
        (xc0, yc1, vx0 * vy1, (1.0 - lx) * ly),
        (xc1, yc1, vx1 * vy1, lx * ly),
    ]
    for t, (xc, yc, v, bw) in enumerate(taps):
        idx_ref[t] = base + (yc * W + xc) * N_HEADS
        w16 = lax.bitcast_convert_type((aw * bw * v).astype(jnp.bfloat16),
                                       jnp.uint16).astype(jnp.uint32)
        wt_ref[t] = lax.bitcast_convert_type((w16 << 16) | w16, jnp.int32)


def _prep(cur2d, posl2d, rp8, Wofp, bofp, Watt, batt):
    grid = ROWS // _PREP_BLK
    row = pl.BlockSpec((_PREP_BLK, D_MODEL), lambda i: (i, 0))
    row8 = pl.BlockSpec((_PREP_BLK, 2 * N_LEVELS), lambda i: (i, 0))
    out4 = pl.BlockSpec((4, _PREP_BLK, NLANE), lambda i: (0, i, 0))
    full = lambda *s: pl.BlockSpec(s, lambda i: (0,) * len(s))
    return pl.pallas_call(
        _prep_body,
        grid=(grid,),
        in_specs=[row, row, row8,
                  full(D_MODEL, D_MODEL), full(D_MODEL),
                  full(D_MODEL, NLANE), full(NLANE),
                  full(2 * N_LEVELS, NLANE), full(2 * N_LEVELS, NLANE),
                  full(NLANE, NLANE)],
        out_specs=(out4, out4),
        out_shape=(jax.ShapeDtypeStruct((4, ROWS, NLANE), jnp.int32),
                   jax.ShapeDtypeStruct((4, ROWS, NLANE), jnp.int32)),
    )(cur2d, posl2d, rp8, Wofp.T, bofp, Watt.T, batt,
      jnp.asarray(_EX), jnp.asarray(_EY), jnp.asarray(_G))


_NC = 2
_NS = 16
_NW = _NC * _NS
_RPW = ROWS // _NW
_SC_CH = 16
_NCH = _RPW // _SC_CH
_NCHP = _NCH // 2
_RING = 8

_GDN = lax.GatherDimensionNumbers(offset_dims=(), collapsed_slice_dims=(0,),
                                  start_index_map=(0,))


def _lane_bcast(v, j):
    idx = jnp.full((16, 1), j, jnp.int32)
    return lax.gather(v, idx, _GDN, (1,),
                      mode=lax.GatherScatterMode.PROMISE_IN_BOUNDS)


def _sc_body(base, nhalf, table, idx, wt, out, idx_v, wt_v, rows_v, out_v,
             sem_i, sem_g):
    rpw = nhalf // _NW
    nchp = rpw // _SC_CH // 2
    wid = lax.axis_index("s") * _NC + lax.axis_index("c")
    out0_w = wid * rpw
    row0_w = base + out0_w

    def chunk_dma(ch, cb):
        row0 = row0_w + ch * _SC_CH
        for t in range(4):
            pltpu.async_copy(idx.at[t, pl.ds(row0, _SC_CH)], idx_v.at[cb, t],
                             sem_i)
            pltpu.async_copy(wt.at[t, pl.ds(row0, _SC_CH)], wt_v.at[cb, t],
                             sem_i)

    def chunk_wait(ch, cb):
        row0 = row0_w + ch * _SC_CH
        for t in range(4):
            pltpu.make_async_copy(idx.at[t, pl.ds(row0, _SC_CH)],
                                  idx_v.at[cb, t], sem_i).wait()
            pltpu.make_async_copy(wt.at[t, pl.ds(row0, _SC_CH)],
                                  wt_v.at[cb, t], sem_i).wait()

    def issue(r, cb, buf):
        for t in range(4):
            pltpu.async_copy(table.at[idx_v.at[cb, t, r]], rows_v.at[buf, t],
                             sem_g.at[buf])

    def wait_gathers(r, cb, buf):
        for t in range(4):
            pltpu.make_async_copy(table.at[idx_v.at[cb, t, r]],
                                  rows_v.at[buf, t], sem_g.at[buf]).wait()

    def reduce_row(r, cb, buf):
        lane2 = 2 * lax.iota(jnp.int32, 16)

        def m_body(m, carry):
            acc_e = jnp.zeros((16,), jnp.float32)
            acc_o = jnp.zeros((16,), jnp.float32)
            for t in range(4):
                wv = wt_v[cb, t, r, pl.ds(m * 16, 16)]
                for j in range(16):
                    wb = plsc.bitcast(_lane_bcast(wv, j), jnp.bfloat16)
                    prod = wb * rows_v[buf, t, m * 16 + j, :]
                    pe, po = plsc.unpack(prod,
                                         format=plsc.PackFormat.INTERLEAVED)
                    acc_e = acc_e + pe
                    acc_o = acc_o + po
            rr = jnp.full((16,), r, jnp.int32)
            plsc.store_scatter(out_v, [rr, m * DH + lane2], acc_e)
            plsc.store_scatter(out_v, [rr, m * DH + lane2 + 1], acc_o)
            return carry
        lax.fori_loop(0, N_HEADS, m_body, 0)

    def chunk_pair_body(pr, carry):
        for cb in range(2):
            ch = 2 * pr + cb
            chunk_wait(ch, cb)
            for rr in range(_RING - 1):
                issue(rr, cb, rr)
            if cb == 0:
                chunk_dma(ch + 1, 1)
            else:
                @pl.when(pr < nchp - 1)
                def _():
                    chunk_dma(ch + 1, 0)

            def row_body(r, c2):
                buf = r & (_RING - 1)
                wait_gathers(r, cb, buf)

                @pl.when(r < _SC_CH - (_RING - 1))
                def _():
                    issue(r + _RING - 1, cb, (r + _RING - 1) & (_RING - 1))
                reduce_row(r, cb, buf)
                return c2

            lax.fori_loop(0, _SC_CH, row_body, 0)
            pltpu.sync_copy(out_v, out.at[pl.ds(out0_w + ch * _SC_CH, _SC_CH)])
        return carry

    chunk_dma(0, 0)
    lax.fori_loop(0, nchp, chunk_pair_body, 0)


@functools.cache
def _sc_attn_fn(base, nhalf):
    return pl.kernel(
        functools.partial(_sc_body, base, nhalf),
        out_type=jax.ShapeDtypeStruct((nhalf, D_MODEL), jnp.float32),
        mesh=plsc.VectorSubcoreMesh(core_axis_name="c", subcore_axis_name="s",
                                    num_cores=_NC, num_subcores=_NS),
        compiler_params=pltpu.CompilerParams(use_tc_tiling_on_sc=False,
                                             needs_layout_passes=False),
        scratch_types=[
            pltpu.VMEM((2, 4, _SC_CH, NLANE), jnp.int32),
            pltpu.VMEM((2, 4, _SC_CH, NLANE), jnp.int32),
            pltpu.VMEM((_RING, 4, NLANE, DH), jnp.bfloat16),
            pltpu.VMEM((_SC_CH, D_MODEL), jnp.float32),
            pltpu.SemaphoreType.DMA,
            pltpu.SemaphoreType.DMA((_RING,)),
        ],
    )


def _sc_attn(table, idx, wt, base, nhalf):
    return _sc_attn_fn(base, nhalf)(table, idx, wt)


_TAIL_BLK = 512


def _tail_body(attn, cur, WoutT, bout, W1T, b1, W2T, b2, g1, be1, g2, be2,
               out_ref):
    src2 = jnp.dot(attn[...], WoutT[...], preferred_element_type=jnp.float32, precision=None)
    x = cur[...] + src2 + bout[...]
    mu = jnp.mean(x, axis=-1, keepdims=True)
    var = jnp.mean((x - mu) ** 2, axis=-1, keepdims=True)
    x = (x - mu) * lax.rsqrt(var + 1e-5) * g1[...] + be1[...]
    h = jnp.dot(x, W1T[...], preferred_element_type=jnp.float32, precision=None) + b1[...]
    h = jnp.maximum(h, 0.0)
    y = jnp.dot(h, W2T[...], preferred_element_type=jnp.float32, precision=None) + b2[...]
    y = x + y
    mu = jnp.mean(y, axis=-1, keepdims=True)
    var = jnp.mean((y - mu) ** 2, axis=-1, keepdims=True)
    out_ref[...] = (y - mu) * lax.rsqrt(var + 1e-5) * g2[...] + be2[...]


def _tail(attn2d, cur2d, Wout, bout, W1, b1, W2, b2, g1, be1, g2, be2):
    nrows = attn2d.shape[0]
    grid = nrows // _TAIL_BLK
    row = pl.BlockSpec((_TAIL_BLK, D_MODEL), lambda i: (i, 0))
    full = lambda *s: pl.BlockSpec(s, lambda i: (0,) * len(s))
    return pl.pallas_call(
        _tail_body,
        grid=(grid,),
        in_specs=[row, row,
                  full(D_MODEL, D_MODEL), full(D_MODEL),
                  full(D_MODEL, D_FFN), full(D_FFN),
                  full(D_FFN, D_MODEL), full(D_MODEL),
                  full(D_MODEL), full(D_MODEL), full(D_MODEL), full(D_MODEL)],
        out_specs=row,
        out_shape=jax.ShapeDtypeStruct((nrows, D_MODEL), jnp.float32),
    )(attn2d, cur2d, Wout.T, bout, W1.T, b1, W2.T, b2, g1, be1, g2, be2)


def kernel(cur_src, src_all, pos, reference_points, spatial_shape,
           Wv, bv, Woff, boff, Watt, batt, Wout, bout,
           W1, b1, W2, b2, g1, be1, g2, be2):
    del spatial_shape
    value = _value(src_all.reshape(B * LIN, D_MODEL), pos, Wv, bv)
    table = value.reshape(B * LIN * N_HEADS, DH)

    perm = jnp.asarray(_PERM)
    idx, wt = _prep(cur_src.reshape(ROWS, D_MODEL),
                    pos[-1].reshape(ROWS, D_MODEL),
                    reference_points.reshape(ROWS, 2 * N_LEVELS),
                    Woff[perm], boff[perm], Watt, batt)

    attn2d = _sc_attn(table, idx, wt, 0, ROWS)
    out = _tail(attn2d, cur_src.reshape(ROWS, D_MODEL),
                Wout, bout, W1, b1, W2, b2, g1, be1, g2, be2)
    return out.reshape(B, LQ, D_MODEL)


# speedup vs baseline: 1.2414x; 1.1048x over previous
import functools

import jax
import jax.numpy as jnp
import numpy as np
from jax import lax
from jax.experimental import pallas as pl
from jax.experimental.pallas import tpu as pltpu
from jax.experimental.pallas import tpu_sc as plsc

B = 2
H = 64
W = 64
N_LEVELS = 4
N_HEADS = 8
N_POINTS = 4
D_MODEL = 256
D_FFN = 1024
LQ = H * W
LIN = N_LEVELS * LQ
ROWS = B * LQ
DH = D_MODEL // N_HEADS
NLANE = N_HEADS * N_LEVELS * N_POINTS

_PERM = np.array([((m * N_LEVELS + l) * N_POINTS + p) * 2 + half
                  for half in range(2)
                  for m in range(N_HEADS)
                  for l in range(N_LEVELS)
                  for p in range(N_POINTS)], dtype=np.int32)

_lanes = np.arange(NLANE)
_lvl = (_lanes % (N_LEVELS * N_POINTS)) // N_POINTS
_EX = np.zeros((2 * N_LEVELS, NLANE), np.float32)
_EX[2 * _lvl, _lanes] = float(W)
_EY = np.zeros((2 * N_LEVELS, NLANE), np.float32)
_EY[2 * _lvl + 1, _lanes] = float(H)
_G = np.kron(np.eye(N_HEADS, dtype=np.float32),
             np.ones((N_LEVELS * N_POINTS, N_LEVELS * N_POINTS), np.float32))

_VAL_BLK = 1024


def _value_body(x, p, WvT, bv, out_ref):
    out_ref[...] = (jnp.dot(x[...] + p[0, 0], WvT[...],
                            preferred_element_type=jnp.float32,
                            precision=None) + bv[...]).astype(jnp.bfloat16)


def _value(src2d, pos4d, Wv, bv):
    n = B * LIN
    nl = LQ // _VAL_BLK
    row = pl.BlockSpec((_VAL_BLK, D_MODEL),
                       lambda b, l, i: (b * N_LEVELS * nl + l * nl + i, 0))
    pspec = pl.BlockSpec((1, 1, _VAL_BLK, D_MODEL), lambda b, l, i: (l, b, i, 0))
    full = lambda *s: pl.BlockSpec(s, lambda b, l, i: (0,) * len(s))
    return pl.pallas_call(
        _value_body,
        grid=(B, N_LEVELS, nl),
        in_specs=[row, pspec, full(D_MODEL, D_MODEL), full(D_MODEL)],
        out_specs=row,
        out_shape=jax.ShapeDtypeStruct((n, D_MODEL), jnp.bfloat16),
    )(src2d, pos4d, Wv.T, bv)


_PREP_BLK = 512


def _prep_body(cur, posl, rp8, WofpT, bofp, WattT, batt, ex, ey, gmat,
               idx_ref, wt_ref):
    q = cur[...] + posl[...]
    w = WofpT[...]
    whi = w.astype(jnp.bfloat16)
    wlo = (w - whi.astype(jnp.float32)).astype(jnp.bfloat16)
    qhi = q.astype(jnp.bfloat16)
    qlo = (q - qhi.astype(jnp.float32)).astype(jnp.bfloat16)
    off = (jnp.dot(qhi, whi, preferred_element_type=jnp.float32)
           + jnp.dot(qhi, wlo, preferred_element_type=jnp.float32)
           + jnp.dot(qlo, whi, preferred_element_type=jnp.float32)
           + bofp[...])
    offx = off[:, :NLANE]
    offy = off[:, NLANE:]
    a = jnp.dot(q, WattT[...], preferred_element_type=jnp.float32) + batt[...]
    a = a - jnp.max(a, axis=-1, keepdims=True)
    e = jnp.exp(a)
    s = jnp.dot(e, gmat[...], preferred_element_type=jnp.float32)
    aw = e / s
    basex = jnp.dot(rp8[...], ex[...], preferred_element_type=jnp.float32, precision=lax.Precision.HIGHEST)
    basey = jnp.dot(rp8[...], ey[...], preferred_element_type=jnp.float32, precision=lax.Precision.HIGHEST)
    x = basex + offx - 0.5
    y = basey + offy - 0.5
    x0f = jnp.floor(x)
    y0f = jnp.floor(y)
    lx = x - x0f
    ly = y - y0f
    lane = lax.broadcasted_iota(jnp.int32, (_PREP_BLK, NLANE), 1)
    m_lane = lane >> 4
    l_lane = (lane >> 2) & 3
    b = pl.program_id(0) * _PREP_BLK // LQ
    base = (b * LIN + l_lane * LQ) * N_HEADS + m_lane
    x0 = x0f.astype(jnp.int32)
    y0 = y0f.astype(jnp.int32)
    x1 = x0 + 1
    y1 = y0 + 1
    vx0 = ((x0 >= 0) & (x0 <= W - 1)).astype(jnp.float32)
    vx1 = ((x1 >= 0) & (x1 <= W - 1)).astype(jnp.float32)
    vy0 = ((y0 >= 0) & (y0 <= H - 1)).astype(jnp.float32)
    vy1 = ((y1 >= 0) & (y1 <= H - 1)).astype(jnp.float32)
    xc0 = jnp.clip(x0, 0, W - 1)
    xc1 = jnp.clip(x1, 0, W - 1)
    yc0 = jnp.clip(y0, 0, H - 1)
    yc1 = jnp.clip(y1, 0, H - 1)
    taps = [
        (xc0, yc0, vx0 * vy0, (1.0 - lx) * (1.0 - ly)),
        (xc1, yc0, vx1 * vy0, lx * (1.0 - ly)),
        (xc0, yc1, vx0 * vy1, (1.0 - lx) * ly),
        (xc1, yc1, vx1 * vy1, lx * ly),
    ]
    for t, (xc, yc, v, bw) in enumerate(taps):
        idx_ref[t] = base + (yc * W + xc) * N_HEADS
        w16 = lax.bitcast_convert_type((aw * bw * v).astype(jnp.bfloat16),
                                       jnp.uint16).astype(jnp.uint32)
        wt_ref[t] = lax.bitcast_convert_type((w16 << 16) | w16, jnp.int32)


def _prep(cur2d, posl2d, rp8, Wofp, bofp, Watt, batt):
    grid = ROWS // _PREP_BLK
    row = pl.BlockSpec((_PREP_BLK, D_MODEL), lambda i: (i, 0))
    row8 = pl.BlockSpec((_PREP_BLK, 2 * N_LEVELS), lambda i: (i, 0))
    out4 = pl.BlockSpec((4, _PREP_BLK, NLANE), lambda i: (0, i, 0))
    full = lambda *s: pl.BlockSpec(s, lambda i: (0,) * len(s))
    return pl.pallas_call(
        _prep_body,
        grid=(grid,),
        in_specs=[row, row, row8,
                  full(D_MODEL, D_MODEL), full(D_MODEL),
                  full(D_MODEL, NLANE), full(NLANE),
                  full(2 * N_LEVELS, NLANE), full(2 * N_LEVELS, NLANE),
                  full(NLANE, NLANE)],
        out_specs=(out4, out4),
        out_shape=(jax.ShapeDtypeStruct((4, ROWS, NLANE), jnp.int32),
                   jax.ShapeDtypeStruct((4, ROWS, NLANE), jnp.int32)),
    )(cur2d, posl2d, rp8, Wofp.T, bofp, Watt.T, batt,
      jnp.asarray(_EX), jnp.asarray(_EY), jnp.asarray(_G))


_NC = 2
_NS = 16
_NW = _NC * _NS
_RPW = ROWS // _NW
_SC_CH = 16
_NCH = _RPW // _SC_CH
_NCHP = _NCH // 2
_RING = 8

_GDN = lax.GatherDimensionNumbers(offset_dims=(), collapsed_slice_dims=(0,),
                                  start_index_map=(0,))


def _lane_bcast(v, j):
    idx = jnp.full((16, 1), j, jnp.int32)
    return lax.gather(v, idx, _GDN, (1,),
                      mode=lax.GatherScatterMode.PROMISE_IN_BOUNDS)


def _sc_body(base, nhalf, table, idx, wt, out, idx_v, wt_v, rows_v, out_v,
             sem_i, sem_g):
    rpw = nhalf // _NW
    nchp = rpw // _SC_CH // 2
    wid = lax.axis_index("s") * _NC + lax.axis_index("c")
    out0_w = wid * rpw
    row0_w = base + out0_w

    def chunk_dma(ch, cb):
        row0 = row0_w + ch * _SC_CH
        for t in range(4):
            pltpu.async_copy(idx.at[t, pl.ds(row0, _SC_CH)], idx_v.at[cb, t],
                             sem_i)
            pltpu.async_copy(wt.at[t, pl.ds(row0, _SC_CH)], wt_v.at[cb, t],
                             sem_i)

    def chunk_wait(ch, cb):
        row0 = row0_w + ch * _SC_CH
        for t in range(4):
            pltpu.make_async_copy(idx.at[t, pl.ds(row0, _SC_CH)],
                                  idx_v.at[cb, t], sem_i).wait()
            pltpu.make_async_copy(wt.at[t, pl.ds(row0, _SC_CH)],
                                  wt_v.at[cb, t], sem_i).wait()

    def issue(r, cb, buf):
        for t in range(4):
            pltpu.async_copy(table.at[idx_v.at[cb, t, r]], rows_v.at[buf, t],
                             sem_g.at[buf])

    def wait_gathers(r, cb, buf):
        for t in range(4):
            pltpu.make_async_copy(table.at[idx_v.at[cb, t, r]],
                                  rows_v.at[buf, t], sem_g.at[buf]).wait()

    def reduce_row(r, cb, buf):
        lane2 = 2 * lax.iota(jnp.int32, 16)

        def m_body(m, carry):
            acc_e = jnp.zeros((16,), jnp.float32)
            acc_o = jnp.zeros((16,), jnp.float32)
            for t in range(4):
                wv = wt_v[cb, t, r, pl.ds(m * 16, 16)]
                for g in range(4):
                    prods = []
                    for j4 in range(4):
                        j = g * 4 + j4
                        wb = plsc.bitcast(_lane_bcast(wv, j), jnp.bfloat16)
                        prods.append(wb * rows_v[buf, t, m * 16 + j, :])
                    s = (prods[0] + prods[1]) + (prods[2] + prods[3])
                    pe, po = plsc.unpack(s,
                                         format=plsc.PackFormat.INTERLEAVED)
                    acc_e = acc_e + pe
                    acc_o = acc_o + po
            rr = jnp.full((16,), r, jnp.int32)
            plsc.store_scatter(out_v, [rr, m * DH + lane2], acc_e)
            plsc.store_scatter(out_v, [rr, m * DH + lane2 + 1], acc_o)
            return carry
        lax.fori_loop(0, N_HEADS, m_body, 0)

    def chunk_pair_body(pr, carry):
        for cb in range(2):
            ch = 2 * pr + cb
            chunk_wait(ch, cb)
            for rr in range(_RING - 1):
                issue(rr, cb, rr)
            if cb == 0:
                chunk_dma(ch + 1, 1)
            else:
                @pl.when(pr < nchp - 1)
                def _():
                    chunk_dma(ch + 1, 0)

            def row_body(r, c2):
                buf = r & (_RING - 1)
                wait_gathers(r, cb, buf)

                @pl.when(r < _SC_CH - (_RING - 1))
                def _():
                    issue(r + _RING - 1, cb, (r + _RING - 1) & (_RING - 1))
                reduce_row(r, cb, buf)
                return c2

            lax.fori_loop(0, _SC_CH, row_body, 0)
            pltpu.sync_copy(out_v, out.at[pl.ds(out0_w + ch * _SC_CH, _SC_CH)])
        return carry

    chunk_dma(0, 0)
    lax.fori_loop(0, nchp, chunk_pair_body, 0)


@functools.cache
def _sc_attn_fn(base, nhalf):
    return pl.kernel(
        functools.partial(_sc_body, base, nhalf),
        out_type=jax.ShapeDtypeStruct((nhalf, D_MODEL), jnp.float32),
        mesh=plsc.VectorSubcoreMesh(core_axis_name="c", subcore_axis_name="s",
                                    num_cores=_NC, num_subcores=_NS),
        compiler_params=pltpu.CompilerParams(use_tc_tiling_on_sc=False,
                                             needs_layout_passes=False),
        scratch_types=[
            pltpu.VMEM((2, 4, _SC_CH, NLANE), jnp.int32),
            pltpu.VMEM((2, 4, _SC_CH, NLANE), jnp.int32),
            pltpu.VMEM((_RING, 4, NLANE, DH), jnp.bfloat16),
            pltpu.VMEM((_SC_CH, D_MODEL), jnp.float32),
            pltpu.SemaphoreType.DMA,
            pltpu.SemaphoreType.DMA((_RING,)),
        ],
    )


def _sc_attn(table, idx, wt, base, nhalf):
    return _sc_attn_fn(base, nhalf)(table, idx, wt)


_TAIL_BLK = 512


def _tail_body(attn, cur, WoutT, bout, W1T, b1, W2T, b2, g1, be1, g2, be2,
               out_ref):
    src2 = jnp.dot(attn[...], WoutT[...], preferred_element_type=jnp.float32, precision=None)
    x = cur[...] + src2 + bout[...]
    mu = jnp.mean(x, axis=-1, keepdims=True)
    var = jnp.mean((x - mu) ** 2, axis=-1, keepdims=True)
    x = (x - mu) * lax.rsqrt(var + 1e-5) * g1[...] + be1[...]
    h = jnp.dot(x, W1T[...], preferred_element_type=jnp.float32, precision=None) + b1[...]
    h = jnp.maximum(h, 0.0)
    y = jnp.dot(h, W2T[...], preferred_element_type=jnp.float32, precision=None) + b2[...]
    y = x + y
    mu = jnp.mean(y, axis=-1, keepdims=True)
    var = jnp.mean((y - mu) ** 2, axis=-1, keepdims=True)
    out_ref[...] = (y - mu) * lax.rsqrt(var + 1e-5) * g2[...] + be2[...]


def _tail(attn2d, cur2d, Wout, bout, W1, b1, W2, b2, g1, be1, g2, be2):
    nrows = attn2d.shape[0]
    grid = nrows // _TAIL_BLK
    row = pl.BlockSpec((_TAIL_BLK, D_MODEL), lambda i: (i, 0))
    full = lambda *s: pl.BlockSpec(s, lambda i: (0,) * len(s))
    return pl.pallas_call(
        _tail_body,
        grid=(grid,),
        in_specs=[row, row,
                  full(D_MODEL, D_MODEL), full(D_MODEL),
                  full(D_MODEL, D_FFN), full(D_FFN),
                  full(D_FFN, D_MODEL), full(D_MODEL),
                  full(D_MODEL), full(D_MODEL), full(D_MODEL), full(D_MODEL)],
        out_specs=row,
        out_shape=jax.ShapeDtypeStruct((nrows, D_MODEL), jnp.float32),
    )(attn2d, cur2d, Wout.T, bout, W1.T, b1, W2.T, b2, g1, be1, g2, be2)


def kernel(cur_src, src_all, pos, reference_points, spatial_shape,
           Wv, bv, Woff, boff, Watt, batt, Wout, bout,
           W1, b1, W2, b2, g1, be1, g2, be2):
    del spatial_shape
    value = _value(src_all.reshape(B * LIN, D_MODEL), pos, Wv, bv)
    table = value.reshape(B * LIN * N_HEADS, DH)

    perm = jnp.asarray(_PERM)
    idx, wt = _prep(cur_src.reshape(ROWS, D_MODEL),
                    pos[-1].reshape(ROWS, D_MODEL),
                    reference_points.reshape(ROWS, 2 * N_LEVELS),
                    Woff[perm], boff[perm], Watt, batt)

    attn2d = _sc_attn(table, idx, wt, 0, ROWS)
    out = _tail(attn2d, cur_src.reshape(ROWS, D_MODEL),
                Wout, bout, W1, b1, W2, b2, g1, be1, g2, be2)
    return out.reshape(B, LQ, D_MODEL)
